# Initial kernel scaffold; baseline (speedup 1.0000x reference)
import jax
import jax.numpy as jnp
from jax.experimental import pallas as pl


def kernel(inputs, mapping):
    raise NotImplementedError("write your pallas kernel here")


# speedup vs baseline: 192.5973x; 192.5973x over previous
import dataclasses
import functools

import jax
import jax.numpy as jnp
from jax import lax
from jax.experimental import pallas as pl
from jax.experimental.pallas import tpu as pltpu
from jax.experimental.pallas import tpu_sc as plsc

NC = 2
NS = 16
L = 16
NW = NC * NS

ROWS, COLS = 4096, 200
TOTAL = ROWS * COLS
CHUNK = TOTAL // NW


@jax.jit
def _sc_lookup(flat, mapping):
    mesh = plsc.VectorSubcoreMesh(
        core_axis_name="c", subcore_axis_name="s",
        num_cores=NC, num_subcores=NS)
    cp = pltpu.CompilerParams()
    if "needs_layout_passes" in pltpu.CompilerParams.__dataclass_fields__:
        cp = dataclasses.replace(cp, needs_layout_passes=False)

    @functools.partial(
        pl.kernel,
        out_type=jax.ShapeDtypeStruct((TOTAL,), jnp.int32),
        mesh=mesh,
        scratch_types=[
            pltpu.VMEM((128,), jnp.int32),
            pltpu.VMEM((CHUNK,), jnp.int32),
            pltpu.VMEM((CHUNK,), jnp.int32),
        ],
        compiler_params=cp,
    )
    def lookup_kernel(flat_hbm, map_hbm, out_hbm, table_v, idx_v, out_v):
        wid = lax.axis_index("s") * NC + lax.axis_index("c")
        base = wid * CHUNK
        pltpu.sync_copy(map_hbm, table_v)
        pltpu.sync_copy(flat_hbm.at[pl.ds(base, CHUNK)], idx_v)

        @pl.loop(0, CHUNK, step=L)
        def _(i):
            idx = idx_v[pl.ds(i, L)]
            out_v[pl.ds(i, L)] = plsc.load_gather(table_v, [idx])

        pltpu.sync_copy(out_v, out_hbm.at[pl.ds(base, CHUNK)])

    return lookup_kernel(flat, mapping)


def kernel(inputs, mapping):
    flat = inputs.reshape(-1)
    return _sc_lookup(flat, mapping).reshape(inputs.shape)
